# baseline (speedup 1.0000x reference)
import functools

import jax
import jax.numpy as jnp
from jax import lax
from jax.experimental import pallas as pl

H = 128
B = 16
N = 16384
NB = 2048


def _prep_body(x_ref, batch_ref, nc_ref, w1_ref, b1_ref, w2_ref, b2_ref,
               qwt_ref, qb_ref, w3_ref, b3_ref, out_ref):
    batch = batch_ref[:, :]
    seg = lax.broadcasted_iota(jnp.int32, (B, N), 0)
    onehot_t = (batch == seg)
    pos = lax.broadcasted_iota(jnp.int32, (B, N), 1)
    masked = jnp.where(onehot_t, pos, -1)
    last = jnp.max(masked, axis=1, keepdims=True)
    last = jnp.clip(last, 0, N - 1)
    lastoh_t = (pos == last).astype(jnp.float32)
    x = x_ref[:, :]
    v_n = jnp.dot(lastoh_t, x, preferred_element_type=jnp.float32)

    c = (jnp.dot(v_n, w1_ref[:, :], preferred_element_type=jnp.float32)
         + b1_ref[:, :] + b2_ref[:, :])
    onehot_f = onehot_t.astype(jnp.float32)
    qb = qb_ref[0, 0]
    w2 = w2_ref[:, :]
    qwt = qwt_ref[:, :]

    s_g = jnp.zeros((B, H), dtype=jnp.float32)
    for k in range(N // NB):
        sl = pl.ds(k * NB, NB)
        xk = x_ref[sl, :]
        oh_k = onehot_f[:, k * NB:(k + 1) * NB]
        cb_k = lax.dot_general(oh_k, c, (((0,), (0,)), ((), ())),
                               preferred_element_type=jnp.float32)
        pre = jnp.dot(xk, w2, preferred_element_type=jnp.float32) + cb_k
        sg = jax.nn.sigmoid(pre)
        alpha = lax.dot_general(qwt, sg, (((1,), (1,)), ((), ())),
                                preferred_element_type=jnp.float32) + qb
        wk = nc_ref[:, k * NB:(k + 1) * NB] * alpha
        a_k = oh_k * wk
        s_g = s_g + jnp.dot(a_k, xk, preferred_element_type=jnp.float32)

    s_h = (jnp.dot(v_n, w3_ref[0:H, :], preferred_element_type=jnp.float32)
           + jnp.dot(s_g, w3_ref[H:2 * H, :], preferred_element_type=jnp.float32)
           + b3_ref[:, :])
    out_ref[:, :] = s_h


def _score_body(s_h_ref, tbl_ref, out_ref):
    out_ref[:, :] = lax.dot_general(
        s_h_ref[:, :], tbl_ref[:, :], (((1,), (1,)), ((), ())),
        preferred_element_type=jnp.float32)


VT = 25088


@jax.jit
def kernel(node_embedding, item_embedding_table, batch, num_count,
           W1, b1, W2, b2, qw, qb, W3, b3):
    n, h = node_embedding.shape
    v = item_embedding_table.shape[0]
    batch_row = batch.astype(jnp.int32).reshape(1, n)
    nc_row = num_count.reshape(1, n)

    s_h = pl.pallas_call(
        _prep_body,
        out_shape=jax.ShapeDtypeStruct((B, h), jnp.float32),
    )(node_embedding, batch_row, nc_row,
      W1, b1.reshape(1, h), W2, b2.reshape(1, h),
      qw.reshape(1, h), qb.reshape(1, 1), W3, b3.reshape(1, h))

    grid = (v + VT - 1) // VT
    z = pl.pallas_call(
        _score_body,
        grid=(grid,),
        in_specs=[
            pl.BlockSpec((B, h), lambda i: (0, 0)),
            pl.BlockSpec((VT, h), lambda i: (i, 0)),
        ],
        out_specs=pl.BlockSpec((B, VT), lambda i: (0, i)),
        out_shape=jax.ShapeDtypeStruct((B, v), jnp.float32),
    )(s_h, item_embedding_table)
    return z


# speedup vs baseline: 1.0463x; 1.0463x over previous
import jax
import jax.numpy as jnp
from jax import lax
from jax.experimental import pallas as pl
from jax.experimental.pallas import tpu as pltpu

H = 128
B = 16
N = 16384
NB = 2048
VT = 12544


def _prep(x_ref, batch_ref, nc_ref, w1_ref, b1_ref, w2_ref, b2_ref,
          qwt_ref, qb_ref, w3_ref, b3_ref, s_h_ref):
    batch = batch_ref[:, :]
    seg = lax.broadcasted_iota(jnp.int32, (B, N), 0)
    onehot_t = (batch == seg)
    pos = lax.broadcasted_iota(jnp.int32, (B, N), 1)
    masked = jnp.where(onehot_t, pos, -1)
    last = jnp.max(masked, axis=1, keepdims=True)
    last = jnp.clip(last, 0, N - 1)
    lastoh_t = (pos == last).astype(jnp.float32)
    x = x_ref[:, :]
    v_n = jnp.dot(lastoh_t, x, preferred_element_type=jnp.float32)

    c = (jnp.dot(v_n, w1_ref[:, :], preferred_element_type=jnp.float32)
         + b1_ref[:, :] + b2_ref[:, :])
    onehot_f = onehot_t.astype(jnp.float32)
    qb = qb_ref[0, 0]
    w2 = w2_ref[:, :]
    qwt = qwt_ref[:, :]

    s_g = jnp.zeros((B, H), dtype=jnp.float32)
    for k in range(N // NB):
        sl = pl.ds(k * NB, NB)
        xk = x_ref[sl, :]
        oh_k = onehot_f[:, k * NB:(k + 1) * NB]
        cb_k = lax.dot_general(oh_k, c, (((0,), (0,)), ((), ())),
                               preferred_element_type=jnp.float32)
        pre = jnp.dot(xk, w2, preferred_element_type=jnp.float32) + cb_k
        sg = jax.nn.sigmoid(pre)
        alpha = lax.dot_general(qwt, sg, (((1,), (1,)), ((), ())),
                                preferred_element_type=jnp.float32) + qb
        wk = nc_ref[:, k * NB:(k + 1) * NB] * alpha
        a_k = oh_k * wk
        s_g = s_g + jnp.dot(a_k, xk, preferred_element_type=jnp.float32)

    s_h_ref[:, :] = (
        jnp.dot(v_n, w3_ref[0:H, :], preferred_element_type=jnp.float32)
        + jnp.dot(s_g, w3_ref[H:2 * H, :], preferred_element_type=jnp.float32)
        + b3_ref[:, :])


def _fused_body(x_ref, batch_ref, nc_ref, w1_ref, b1_ref, w2_ref, b2_ref,
                qwt_ref, qb_ref, w3_ref, b3_ref, tbl_ref, out_ref, s_h_ref):
    @pl.when(pl.program_id(0) == 0)
    def _():
        _prep(x_ref, batch_ref, nc_ref, w1_ref, b1_ref, w2_ref, b2_ref,
              qwt_ref, qb_ref, w3_ref, b3_ref, s_h_ref)

    out_ref[:, :] = lax.dot_general(
        s_h_ref[:, :], tbl_ref[:, :], (((1,), (1,)), ((), ())),
        preferred_element_type=jnp.float32)


@jax.jit
def kernel(node_embedding, item_embedding_table, batch, num_count,
           W1, b1, W2, b2, qw, qb, W3, b3):
    n, h = node_embedding.shape
    v = item_embedding_table.shape[0]
    batch_row = batch.astype(jnp.int32).reshape(1, n)
    nc_row = num_count.reshape(1, n)

    const = lambda i: (0, 0)
    grid = (v + VT - 1) // VT
    z = pl.pallas_call(
        _fused_body,
        grid=(grid,),
        in_specs=[
            pl.BlockSpec((n, h), const),
            pl.BlockSpec((1, n), const),
            pl.BlockSpec((1, n), const),
            pl.BlockSpec((h, h), const),
            pl.BlockSpec((1, h), const),
            pl.BlockSpec((h, h), const),
            pl.BlockSpec((1, h), const),
            pl.BlockSpec((1, h), const),
            pl.BlockSpec((1, 1), const),
            pl.BlockSpec((2 * h, h), const),
            pl.BlockSpec((1, h), const),
            pl.BlockSpec((VT, h), lambda i: (i, 0)),
        ],
        out_specs=pl.BlockSpec((B, VT), lambda i: (0, i)),
        out_shape=jax.ShapeDtypeStruct((B, v), jnp.float32),
        scratch_shapes=[pltpu.VMEM((B, h), jnp.float32)],
    )(node_embedding, batch_row, nc_row,
      W1, b1.reshape(1, h), W2, b2.reshape(1, h),
      qw.reshape(1, h), qb.reshape(1, 1), W3, b3.reshape(1, h),
      item_embedding_table)
    return z
